# baseline (speedup 1.0000x reference)
import functools

import jax
import jax.numpy as jnp
from jax import lax
from jax.experimental import pallas as pl
from jax.experimental.pallas import tpu as pltpu
from jax.experimental.pallas import tpu_sc as plsc

N = 100000
D = 128
B = 16384
B2 = 2 * B

NC = 2
NS = 16
NW = NC * NS
L = 16

BPW = B // NW
RPW = 3128
RPW_LAST = N - RPW * (NW - 1)
LIST_CH = 26
LIST_LEN = LIST_CH * 128
SENT = 0x7FFFFFFF

BM = 1024

_mesh = functools.partial(
    plsc.VectorSubcoreMesh, core_axis_name="c", subcore_axis_name="s",
    num_cores=NC, num_subcores=NS)


def _wid():
    return lax.axis_index("s") * NC + lax.axis_index("c")


def _iota16():
    return lax.iota(jnp.int32, L)


def _splat(x, dtype=jnp.int32):
    return jnp.full((L,), x, dtype=dtype)


def _first_lane(vec):
    return jnp.max(jnp.where(_iota16() == 0, vec, jnp.int32(-2147483648)))


def _k1_body(mem_hbm, lu_hbm, src_hbm, dst_hbm, t_hbm,
             mem_src_out, mem_dst_out, lu_src_out, lu_dst_out,
             nodes_out, posw_out, counts_out, lu_out,
             all_idx, rows_v, lu_v, prio, nodes2d, posw2d, cnt128,
             t_v, lu_rng,
             sem_g, sem_lu):
    wid = _wid()
    iota = _iota16()
    base_pos = wid * BPW
    HW = BPW // 2

    pltpu.sync_copy(src_hbm, all_idx.at[pl.ds(0, B)])
    pltpu.sync_copy(dst_hbm, all_idx.at[pl.ds(B, B)])
    pltpu.sync_copy(t_hbm, t_v)

    gather = pltpu.async_copy(
        mem_hbm.at[all_idx.at[pl.ds(base_pos, HW)]], rows_v, sem_g)

    base_node = wid * RPW
    my_start = base_node
    my_len = jnp.where(wid == NW - 1, RPW_LAST, RPW)

    def init_prio(j, _):
        prio[pl.ds(j * L, L)] = _splat(-1)
        return 0
    lax.fori_loop(0, (RPW + 8 + L - 1) // L, init_prio, 0)

    def scan_step(v, _):
        idxv = all_idx[pl.ds(v * L, L)]
        loc = idxv - base_node
        inr = (loc >= 0) & (loc < my_len)
        _, last = plsc.scan_count(loc, mask=inr)
        win = last & inr
        pos = v * L + iota
        loc_safe = jnp.where(win, loc, 0)
        plsc.store_scatter(prio, [loc_safe], pos, mask=win)
        return 0

    lax.fori_loop(0, B2 // L, scan_step, 0)

    TAIL = RPW - RPW_LAST
    tail = my_len - TAIL
    pltpu.sync_copy(lu_hbm.at[pl.ds(my_start, RPW_LAST)],
                    lu_rng.at[pl.ds(0, RPW_LAST)])
    pltpu.sync_copy(lu_hbm.at[pl.ds(my_start + tail, TAIL)],
                    lu_rng.at[pl.ds(tail, TAIL)])

    def lu_step(j, _):
        pr = prio[pl.ds(j * L, L)]
        m = pr >= 0
        pm = pr & (B - 1)
        tv = plsc.load_gather(t_v, [pm])
        cur = lu_rng[pl.ds(j * L, L)]
        lu_rng[pl.ds(j * L, L)] = jnp.where(m, tv, cur)
        return 0
    lax.fori_loop(0, (RPW + 8) // L, lu_step, 0)

    pltpu.sync_copy(lu_rng.at[pl.ds(0, RPW_LAST)],
                    lu_out.at[pl.ds(my_start, RPW_LAST)])
    pltpu.sync_copy(lu_rng.at[pl.ds(tail, TAIL)],
                    lu_out.at[pl.ds(my_start + tail, TAIL)])

    def init_lists(j, _):
        r = j >> 3
        c = (j & 7) * L
        nodes2d[r, pl.ds(c, L)] = _splat(0)
        posw2d[r, pl.ds(c, L)] = _splat(0)
        return 0
    lax.fori_loop(0, LIST_CH * 8, init_lists, 0)

    def compact_step(j, cursor):
        pr = prio[pl.ds(j * L, L)]
        valid = pr >= 0
        node = base_node + j * L + iota
        incl = plsc.cumsum(jnp.where(valid, 1, 0))
        slot = cursor + incl - 1
        slot = jnp.maximum(slot, 0)
        srow = slot >> 7
        scol = slot & 127
        plsc.store_scatter(nodes2d, [srow, scol], node, mask=valid)
        plsc.store_scatter(posw2d, [srow, scol], pr, mask=valid)
        return cursor + jnp.max(incl)

    cnt = lax.fori_loop(0, (RPW + 8) // L, compact_step, jnp.int32(0))

    @pl.when(cnt > 0)
    def _fill():
        node0 = _first_lane(nodes2d[0, pl.ds(0, L)])
        pos0 = _first_lane(posw2d[0, pl.ds(0, L)])

        def fill_step(j, _):
            r = j >> 3
            c = (j & 7) * L
            slot = j * L + iota
            m = slot >= cnt
            cur_n = nodes2d[r, pl.ds(c, L)]
            cur_p = posw2d[r, pl.ds(c, L)]
            nodes2d[r, pl.ds(c, L)] = jnp.where(m, node0, cur_n)
            posw2d[r, pl.ds(c, L)] = jnp.where(m, pos0, cur_p)
            return 0
        lax.fori_loop(0, LIST_CH * 8, fill_step, 0)

    gather.wait()
    pltpu.sync_copy(rows_v, mem_src_out.at[pl.ds(base_pos, HW)])
    for idx_off, out, out_off in (
            (base_pos + HW, mem_src_out, base_pos + HW),
            (B + base_pos, mem_dst_out, base_pos),
            (B + base_pos + HW, mem_dst_out, base_pos + HW)):
        pltpu.async_copy(mem_hbm.at[all_idx.at[pl.ds(idx_off, HW)]],
                         rows_v, sem_g).wait()
        pltpu.sync_copy(rows_v, out.at[pl.ds(out_off, HW)])

    src_idx = all_idx.at[pl.ds(base_pos, BPW)]
    dst_idx = all_idx.at[pl.ds(B + base_pos, BPW)]
    pltpu.async_copy(lu_hbm.at[src_idx], lu_v, sem_lu).wait()
    pltpu.sync_copy(lu_v, lu_src_out.at[pl.ds(base_pos, BPW)])
    pltpu.async_copy(lu_hbm.at[dst_idx], lu_v, sem_lu).wait()
    pltpu.sync_copy(lu_v, lu_dst_out.at[pl.ds(base_pos, BPW)])

    pltpu.sync_copy(nodes2d, nodes_out.at[wid])
    pltpu.sync_copy(posw2d, posw_out.at[wid])
    counts_v = jnp.full((L,), cnt, dtype=jnp.int32)
    for l in range(8):
        cnt128[pl.ds(l * L, L)] = counts_v
    pltpu.sync_copy(cnt128, counts_out.at[wid])


def _k1(memory, last_update, src, dst, t):
    out_type = (
        jax.ShapeDtypeStruct((B, D), jnp.float32),
        jax.ShapeDtypeStruct((B, D), jnp.float32),
        jax.ShapeDtypeStruct((B,), jnp.float32),
        jax.ShapeDtypeStruct((B,), jnp.float32),
        jax.ShapeDtypeStruct((NW, LIST_CH, 128), jnp.int32),
        jax.ShapeDtypeStruct((NW, LIST_CH, 128), jnp.int32),
        jax.ShapeDtypeStruct((NW, 128), jnp.int32),
        jax.ShapeDtypeStruct((N,), jnp.float32),
    )
    scratch = [
        pltpu.VMEM((B2,), jnp.int32),
        pltpu.VMEM((BPW // 2, D), jnp.float32),
        pltpu.VMEM((BPW,), jnp.float32),
        pltpu.VMEM((RPW + L, ), jnp.int32),
        pltpu.VMEM((LIST_CH, 128), jnp.int32),
        pltpu.VMEM((LIST_CH, 128), jnp.int32),
        pltpu.VMEM((128,), jnp.int32),
        pltpu.VMEM((B,), jnp.float32),
        pltpu.VMEM((RPW + 8,), jnp.float32),
        pltpu.SemaphoreType.DMA,
        pltpu.SemaphoreType.DMA,
    ]
    return pl.kernel(
        _k1_body, out_type=out_type, mesh=_mesh(), scratch_types=scratch,
        compiler_params=pltpu.CompilerParams(needs_layout_passes=False),
    )(memory, last_update, src, dst, t)


_INV_2PI = 0.15915494309189535
_CW1 = 6.28125
_CW2 = 0.0019353071795864769
_COS_POLY = (1.0, -0.49999985098838806, 0.041666463017463684,
             -0.0013887732056900859, 2.4769053197815083e-05,
             -2.707544979330123e-07, 1.7243751981865785e-09)


def _fast_cos(x):
    n = jnp.round(x * _INV_2PI)
    r = x - n * _CW1 - n * _CW2
    u = r * r
    acc = jnp.float32(_COS_POLY[-1])
    for c in _COS_POLY[-2::-1]:
        acc = acc * u + jnp.float32(c)
    return acc


def _dense_body(t_ref, lus_ref, lud_ref, ms_ref, md_ref, ef_ref,
                tw_ref, tb_ref, w_own_ref, w_oth_ref, w_te_ref, w_ef_ref,
                w_hh_ref, bi_ref, bh_ref, out_ref):
    is_src = pl.program_id(1) == 0
    ms = ms_ref[...]
    md = md_ref[...]
    own = jnp.where(is_src, ms, md)
    oth = jnp.where(is_src, md, ms)
    lu = jnp.where(is_src, lus_ref[...], lud_ref[...])
    te = _fast_cos((t_ref[...] - lu) * tw_ref[...] + tb_ref[...])

    bf = jnp.bfloat16
    gi = (jnp.dot(own.astype(bf), w_own_ref[...], preferred_element_type=jnp.float32)
          + jnp.dot(oth.astype(bf), w_oth_ref[...], preferred_element_type=jnp.float32)
          + jnp.dot(te.astype(bf), w_te_ref[...], preferred_element_type=jnp.float32)
          + jnp.dot(ef_ref[...].astype(bf), w_ef_ref[...],
                    preferred_element_type=jnp.float32)
          + bi_ref[...])
    gh = jnp.dot(own.astype(bf), w_hh_ref[...], preferred_element_type=jnp.float32) \
        + bh_ref[...]
    r = jax.nn.sigmoid(gi[:, :D] + gh[:, :D])
    z = jax.nn.sigmoid(gi[:, D:2 * D] + gh[:, D:2 * D])
    n = jnp.tanh(gi[:, 2 * D:] + r * gh[:, 2 * D:])
    out_ref[...] = (1.0 - z) * n + z * own


def _dense_update(t2, lu_src, lu_dst, mem_src, mem_dst, edge_feat,
                  time_w, time_b, W_ih, W_hh, b_ih, b_hh):
    W_ihT = W_ih.T.astype(jnp.bfloat16)
    w_own = W_ihT[0:D]
    w_oth = W_ihT[D:2 * D]
    w_te = W_ihT[2 * D:3 * D]
    w_ef = W_ihT[3 * D:]
    w_hh = W_hh.T.astype(jnp.bfloat16)
    bi = b_ih.reshape(1, -1)
    bh = b_hh.reshape(1, -1)
    tw = time_w.reshape(1, -1)
    tb = time_b.reshape(1, -1)

    grid = (B // BM, 2)
    row_blk = pl.BlockSpec((BM, 1), lambda i, j: (i, 0))
    mat_blk = pl.BlockSpec((BM, D), lambda i, j: (i, 0))

    def full(a):
        return pl.BlockSpec(a.shape, lambda i, j: tuple(0 for _ in a.shape))

    out_spec = pl.BlockSpec((BM, D), lambda i, j: (j * (B // BM) + i, 0))
    return pl.pallas_call(
        _dense_body,
        grid=grid,
        in_specs=[row_blk, row_blk, row_blk, mat_blk, mat_blk, mat_blk,
                  full(tw), full(tb), full(w_own), full(w_oth), full(w_te),
                  full(w_ef), full(w_hh), full(bi), full(bh)],
        out_specs=out_spec,
        out_shape=jax.ShapeDtypeStruct((B2, D), jnp.float32),
    )(t2, lu_src, lu_dst, mem_src, mem_dst, edge_feat,
      tw, tb, w_own, w_oth, w_te, w_ef, w_hh, bi, bh)


def _k3_body(u_hbm, nodes_hbm, posw_hbm, counts_hbm,
             mem_ref,
             nodes2d, posw2d, rowbuf0, rowbuf1, tmp16,
             sem_ga, sem_gb, sem_sa, sem_sb):
    wid = _wid()

    pltpu.sync_copy(counts_hbm.at[wid], tmp16)
    cnt = jnp.max(tmp16[pl.ds(0, L)])
    pltpu.sync_copy(nodes_hbm.at[wid], nodes2d)
    pltpu.sync_copy(posw_hbm.at[wid], posw2d)

    @pl.when(cnt > 0)
    def _scatter():
        nb_ch = (cnt + 127) >> 7

        def sc_step(i, _):
            ca = 2 * i
            cb = jnp.minimum(2 * i + 1, nb_ch - 1)
            ga = pltpu.async_copy(u_hbm.at[posw2d.at[ca]], rowbuf0, sem_ga)
            gb = pltpu.async_copy(u_hbm.at[posw2d.at[cb]], rowbuf1, sem_gb)
            ga.wait()
            sa = pltpu.async_copy(rowbuf0, mem_ref.at[nodes2d.at[ca]], sem_sa)
            gb.wait()
            sb = pltpu.async_copy(rowbuf1, mem_ref.at[nodes2d.at[cb]], sem_sb)
            sa.wait()
            sb.wait()
            return 0

        lax.fori_loop(0, (nb_ch + 1) >> 1, sc_step, 0)


def _k3(u, nodes, posw, counts, mem_ref):
    scratch = [
        pltpu.VMEM((LIST_CH, 128), jnp.int32),
        pltpu.VMEM((LIST_CH, 128), jnp.int32),
        pltpu.VMEM((128, D), jnp.float32),
        pltpu.VMEM((128, D), jnp.float32),
        pltpu.VMEM((128,), jnp.int32),
    ] + [pltpu.SemaphoreType.DMA] * 4
    return pl.kernel(
        _k3_body, out_type=(), mesh=_mesh(), scratch_types=scratch,
        compiler_params=pltpu.CompilerParams(needs_layout_passes=False),
    )(u, nodes, posw, counts, mem_ref)


def kernel(memory, last_update, src, dst, t, edge_feat, time_w, time_b,
           W_ih, W_hh, b_ih, b_hh):
    (mem_src, mem_dst, lu_src, lu_dst, nodes, posw, counts,
     new_last_update) = _k1(memory, last_update, src, dst, t)

    u = _dense_update(t.reshape(B, 1), lu_src.reshape(B, 1),
                      lu_dst.reshape(B, 1), mem_src, mem_dst,
                      edge_feat, time_w, time_b, W_ih, W_hh, b_ih, b_hh)

    mem_ref = jax.new_ref(memory)
    _k3(u, nodes, posw, counts, mem_ref)
    return (mem_ref[...], new_last_update)


# speedup vs baseline: 1.0013x; 1.0013x over previous
import functools

import jax
import jax.numpy as jnp
from jax import lax
from jax.experimental import pallas as pl
from jax.experimental.pallas import tpu as pltpu
from jax.experimental.pallas import tpu_sc as plsc

N = 100000
D = 128
B = 16384
B2 = 2 * B

NC = 2
NS = 16
NW = NC * NS
L = 16

BPW = B // NW
RPW = 3128
RPW_LAST = N - RPW * (NW - 1)
LIST_CH = 26
LIST_LEN = LIST_CH * 128
SENT = 0x7FFFFFFF

BM = 1024

_mesh = functools.partial(
    plsc.VectorSubcoreMesh, core_axis_name="c", subcore_axis_name="s",
    num_cores=NC, num_subcores=NS)


def _wid():
    return lax.axis_index("s") * NC + lax.axis_index("c")


def _iota16():
    return lax.iota(jnp.int32, L)


def _splat(x, dtype=jnp.int32):
    return jnp.full((L,), x, dtype=dtype)


def _first_lane(vec):
    return jnp.max(jnp.where(_iota16() == 0, vec, jnp.int32(-2147483648)))


def _k1_body(mem_hbm, lu_hbm, src_hbm, dst_hbm, t_hbm,
             mem_src_out, mem_dst_out, lu_src_out, lu_dst_out,
             nodes_out, posw_out, counts_out, lu_out,
             all_idx, rows_v, lu_v, prio, nodes2d, posw2d, cnt128,
             t_v, lu_rng,
             sem_g, sem_lu):
    wid = _wid()
    iota = _iota16()
    base_pos = wid * BPW
    HW = BPW // 2

    pltpu.sync_copy(src_hbm, all_idx.at[pl.ds(0, B)])
    pltpu.sync_copy(dst_hbm, all_idx.at[pl.ds(B, B)])
    pltpu.sync_copy(t_hbm, t_v)

    gather = pltpu.async_copy(
        mem_hbm.at[all_idx.at[pl.ds(base_pos, HW)]], rows_v, sem_g)

    base_node = wid * RPW
    my_start = base_node
    my_len = jnp.where(wid == NW - 1, RPW_LAST, RPW)

    def init_prio(j, _):
        prio[pl.ds(j * L, L)] = _splat(-1)
        return 0
    lax.fori_loop(0, (RPW + 8 + L - 1) // L, init_prio, 0)

    def scan_step(v, _):
        idxv = all_idx[pl.ds(v * L, L)]
        loc = idxv - base_node
        inr = (loc >= 0) & (loc < my_len)
        _, last = plsc.scan_count(loc, mask=inr)
        win = last & inr
        pos = v * L + iota
        loc_safe = jnp.where(win, loc, 0)
        plsc.store_scatter(prio, [loc_safe], pos, mask=win)
        return 0

    lax.fori_loop(0, B2 // L, scan_step, 0)

    TAIL = RPW - RPW_LAST
    tail = my_len - TAIL
    pltpu.sync_copy(lu_hbm.at[pl.ds(my_start, RPW_LAST)],
                    lu_rng.at[pl.ds(0, RPW_LAST)])
    pltpu.sync_copy(lu_hbm.at[pl.ds(my_start + tail, TAIL)],
                    lu_rng.at[pl.ds(tail, TAIL)])

    def lu_step(j, _):
        pr = prio[pl.ds(j * L, L)]
        m = pr >= 0
        pm = pr & (B - 1)
        tv = plsc.load_gather(t_v, [pm])
        cur = lu_rng[pl.ds(j * L, L)]
        lu_rng[pl.ds(j * L, L)] = jnp.where(m, tv, cur)
        return 0
    lax.fori_loop(0, (RPW + 8) // L, lu_step, 0)

    pltpu.sync_copy(lu_rng.at[pl.ds(0, RPW_LAST)],
                    lu_out.at[pl.ds(my_start, RPW_LAST)])
    pltpu.sync_copy(lu_rng.at[pl.ds(tail, TAIL)],
                    lu_out.at[pl.ds(my_start + tail, TAIL)])

    def init_lists(j, _):
        r = j >> 3
        c = (j & 7) * L
        nodes2d[r, pl.ds(c, L)] = _splat(0)
        posw2d[r, pl.ds(c, L)] = _splat(0)
        return 0
    lax.fori_loop(0, LIST_CH * 8, init_lists, 0)

    def compact_step(j, cursor):
        pr = prio[pl.ds(j * L, L)]
        valid = pr >= 0
        node = base_node + j * L + iota
        incl = plsc.cumsum(jnp.where(valid, 1, 0))
        slot = cursor + incl - 1
        slot = jnp.maximum(slot, 0)
        srow = slot >> 7
        scol = slot & 127
        plsc.store_scatter(nodes2d, [srow, scol], node, mask=valid)
        plsc.store_scatter(posw2d, [srow, scol], pr, mask=valid)
        return cursor + jnp.max(incl)

    cnt = lax.fori_loop(0, (RPW + 8) // L, compact_step, jnp.int32(0))

    @pl.when(cnt > 0)
    def _fill():
        node0 = _first_lane(nodes2d[0, pl.ds(0, L)])
        pos0 = _first_lane(posw2d[0, pl.ds(0, L)])

        def fill_step(j, _):
            r = j >> 3
            c = (j & 7) * L
            slot = j * L + iota
            m = slot >= cnt
            cur_n = nodes2d[r, pl.ds(c, L)]
            cur_p = posw2d[r, pl.ds(c, L)]
            nodes2d[r, pl.ds(c, L)] = jnp.where(m, node0, cur_n)
            posw2d[r, pl.ds(c, L)] = jnp.where(m, pos0, cur_p)
            return 0
        lax.fori_loop(0, LIST_CH * 8, fill_step, 0)

    gather.wait()
    pltpu.sync_copy(rows_v, mem_src_out.at[pl.ds(base_pos, HW)])
    for idx_off, out, out_off in (
            (base_pos + HW, mem_src_out, base_pos + HW),
            (B + base_pos, mem_dst_out, base_pos),
            (B + base_pos + HW, mem_dst_out, base_pos + HW)):
        pltpu.async_copy(mem_hbm.at[all_idx.at[pl.ds(idx_off, HW)]],
                         rows_v, sem_g).wait()
        pltpu.sync_copy(rows_v, out.at[pl.ds(out_off, HW)])

    src_idx = all_idx.at[pl.ds(base_pos, BPW)]
    dst_idx = all_idx.at[pl.ds(B + base_pos, BPW)]
    pltpu.async_copy(lu_hbm.at[src_idx], lu_v, sem_lu).wait()
    pltpu.sync_copy(lu_v, lu_src_out.at[pl.ds(base_pos, BPW)])
    pltpu.async_copy(lu_hbm.at[dst_idx], lu_v, sem_lu).wait()
    pltpu.sync_copy(lu_v, lu_dst_out.at[pl.ds(base_pos, BPW)])

    pltpu.sync_copy(nodes2d, nodes_out.at[wid])
    pltpu.sync_copy(posw2d, posw_out.at[wid])
    counts_v = jnp.full((L,), cnt, dtype=jnp.int32)
    for l in range(8):
        cnt128[pl.ds(l * L, L)] = counts_v
    pltpu.sync_copy(cnt128, counts_out.at[wid])


def _k1(memory, last_update, src, dst, t):
    out_type = (
        jax.ShapeDtypeStruct((B, D), jnp.float32),
        jax.ShapeDtypeStruct((B, D), jnp.float32),
        jax.ShapeDtypeStruct((B,), jnp.float32),
        jax.ShapeDtypeStruct((B,), jnp.float32),
        jax.ShapeDtypeStruct((NW, LIST_CH, 128), jnp.int32),
        jax.ShapeDtypeStruct((NW, LIST_CH, 128), jnp.int32),
        jax.ShapeDtypeStruct((NW, 128), jnp.int32),
        jax.ShapeDtypeStruct((N,), jnp.float32),
    )
    scratch = [
        pltpu.VMEM((B2,), jnp.int32),
        pltpu.VMEM((BPW // 2, D), jnp.float32),
        pltpu.VMEM((BPW,), jnp.float32),
        pltpu.VMEM((RPW + L, ), jnp.int32),
        pltpu.VMEM((LIST_CH, 128), jnp.int32),
        pltpu.VMEM((LIST_CH, 128), jnp.int32),
        pltpu.VMEM((128,), jnp.int32),
        pltpu.VMEM((B,), jnp.float32),
        pltpu.VMEM((RPW + 8,), jnp.float32),
        pltpu.SemaphoreType.DMA,
        pltpu.SemaphoreType.DMA,
    ]
    return pl.kernel(
        _k1_body, out_type=out_type, mesh=_mesh(), scratch_types=scratch,
        compiler_params=pltpu.CompilerParams(needs_layout_passes=False),
    )(memory, last_update, src, dst, t)


_INV_2PI = 0.15915494309189535
_CW1 = 6.28125
_CW2 = 0.0019353071795864769
_COS_POLY = (1.0, -0.49999985098838806, 0.041666463017463684,
             -0.0013887732056900859, 2.4769053197815083e-05,
             -2.707544979330123e-07, 1.7243751981865785e-09)


def _fast_cos(x):
    n = jnp.round(x * _INV_2PI)
    r = x - n * _CW1 - n * _CW2
    u = r * r
    acc = jnp.float32(_COS_POLY[-1])
    for c in _COS_POLY[-2::-1]:
        acc = acc * u + jnp.float32(c)
    return acc


def _dense_body(t_ref, lus_ref, lud_ref, ms_ref, md_ref, ef_ref,
                tw_ref, tb_ref, w_own_ref, w_oth_ref, w_te_ref, w_ef_ref,
                w_hh_ref, bi_ref, bh_ref, out_ref):
    is_src = pl.program_id(1) == 0
    ms = ms_ref[...]
    md = md_ref[...]
    own = jnp.where(is_src, ms, md)
    oth = jnp.where(is_src, md, ms)
    lu = jnp.where(is_src, lus_ref[...], lud_ref[...])
    te = _fast_cos((t_ref[...] - lu) * tw_ref[...] + tb_ref[...])

    bf = jnp.bfloat16
    gi = (jnp.dot(own.astype(bf), w_own_ref[...], preferred_element_type=jnp.float32)
          + jnp.dot(oth.astype(bf), w_oth_ref[...], preferred_element_type=jnp.float32)
          + jnp.dot(te.astype(bf), w_te_ref[...], preferred_element_type=jnp.float32)
          + jnp.dot(ef_ref[...].astype(bf), w_ef_ref[...],
                    preferred_element_type=jnp.float32)
          + bi_ref[...])
    gh = jnp.dot(own.astype(bf), w_hh_ref[...], preferred_element_type=jnp.float32) \
        + bh_ref[...]
    r = jax.nn.sigmoid(gi[:, :D] + gh[:, :D])
    z = jax.nn.sigmoid(gi[:, D:2 * D] + gh[:, D:2 * D])
    n = jnp.tanh(gi[:, 2 * D:] + r * gh[:, 2 * D:])
    out_ref[...] = (1.0 - z) * n + z * own


def _dense_update(t2, lu_src, lu_dst, mem_src, mem_dst, edge_feat,
                  time_w, time_b, W_ih, W_hh, b_ih, b_hh):
    W_ihT = W_ih.T.astype(jnp.bfloat16)
    w_own = W_ihT[0:D]
    w_oth = W_ihT[D:2 * D]
    w_te = W_ihT[2 * D:3 * D]
    w_ef = W_ihT[3 * D:]
    w_hh = W_hh.T.astype(jnp.bfloat16)
    bi = b_ih.reshape(1, -1)
    bh = b_hh.reshape(1, -1)
    tw = time_w.reshape(1, -1)
    tb = time_b.reshape(1, -1)

    grid = (B // BM, 2)
    row_blk = pl.BlockSpec((BM, 1), lambda i, j: (i, 0))
    mat_blk = pl.BlockSpec((BM, D), lambda i, j: (i, 0))

    def full(a):
        return pl.BlockSpec(a.shape, lambda i, j: tuple(0 for _ in a.shape))

    out_spec = pl.BlockSpec((BM, D), lambda i, j: (j * (B // BM) + i, 0))
    return pl.pallas_call(
        _dense_body,
        grid=grid,
        in_specs=[row_blk, row_blk, row_blk, mat_blk, mat_blk, mat_blk,
                  full(tw), full(tb), full(w_own), full(w_oth), full(w_te),
                  full(w_ef), full(w_hh), full(bi), full(bh)],
        out_specs=out_spec,
        out_shape=jax.ShapeDtypeStruct((B2, D), jnp.float32),
    )(t2, lu_src, lu_dst, mem_src, mem_dst, edge_feat,
      tw, tb, w_own, w_oth, w_te, w_ef, w_hh, bi, bh)


def _k3_body(u_hbm, nodes_hbm, posw_hbm, counts_hbm,
             mem_ref,
             nodes2d, posw2d, rowbuf0, rowbuf1, tmp16,
             sem_ga, sem_gb, sem_sa, sem_sb):
    wid = _wid()

    pltpu.sync_copy(counts_hbm.at[wid], tmp16)
    cnt = jnp.max(tmp16[pl.ds(0, L)])
    pltpu.sync_copy(nodes_hbm.at[wid], nodes2d)
    pltpu.sync_copy(posw_hbm.at[wid], posw2d)

    @pl.when(cnt > 0)
    def _scatter():
        nb_ch = (cnt + 127) >> 7

        def sc_step(i, _):
            ca = 2 * i
            cb = jnp.minimum(2 * i + 1, nb_ch - 1)
            ga = pltpu.async_copy(u_hbm.at[posw2d.at[ca]], rowbuf0, sem_ga)
            gb = pltpu.async_copy(u_hbm.at[posw2d.at[cb]], rowbuf1, sem_gb)
            ga.wait()
            sa = pltpu.async_copy(rowbuf0, mem_ref.at[nodes2d.at[ca]], sem_sa)
            gb.wait()
            sb = pltpu.async_copy(rowbuf1, mem_ref.at[nodes2d.at[cb]], sem_sb)
            sa.wait()
            sb.wait()
            return 0

        lax.fori_loop(0, (nb_ch + 1) >> 1, sc_step, 0)


def _k3(u, nodes, posw, counts, mem_ref):
    scratch = [
        pltpu.VMEM((LIST_CH, 128), jnp.int32),
        pltpu.VMEM((LIST_CH, 128), jnp.int32),
        pltpu.VMEM((128, D), jnp.float32),
        pltpu.VMEM((128, D), jnp.float32),
        pltpu.VMEM((128,), jnp.int32),
    ] + [pltpu.SemaphoreType.DMA] * 4
    return pl.kernel(
        _k3_body, out_type=(), mesh=_mesh(), scratch_types=scratch,
        compiler_params=pltpu.CompilerParams(needs_layout_passes=False),
    )(u, nodes, posw, counts, mem_ref)


def kernel(memory, last_update, src, dst, t, edge_feat, time_w, time_b,
           W_ih, W_hh, b_ih, b_hh):
    mem_ref = jax.new_ref(memory)
    (mem_src, mem_dst, lu_src, lu_dst, nodes, posw, counts,
     new_last_update) = _k1(memory, last_update, src, dst, t)

    u = _dense_update(t.reshape(B, 1), lu_src.reshape(B, 1),
                      lu_dst.reshape(B, 1), mem_src, mem_dst,
                      edge_feat, time_w, time_b, W_ih, W_hh, b_ih, b_hh)

    _k3(u, nodes, posw, counts, mem_ref)
    return (mem_ref[...], new_last_update)


# speedup vs baseline: 1.0324x; 1.0311x over previous
import functools

import jax
import jax.numpy as jnp
from jax import lax
from jax.experimental import pallas as pl
from jax.experimental.pallas import tpu as pltpu
from jax.experimental.pallas import tpu_sc as plsc

N = 100000
D = 128
B = 16384
B2 = 2 * B

NC = 2
NS = 16
NW = NC * NS
L = 16

BPW = B // NW
RPW = 3128
RPW_LAST = N - RPW * (NW - 1)
LIST_CH = 26
LIST_LEN = LIST_CH * 128
SENT = 0x7FFFFFFF

BM = 1024

_mesh = functools.partial(
    plsc.VectorSubcoreMesh, core_axis_name="c", subcore_axis_name="s",
    num_cores=NC, num_subcores=NS)


def _wid():
    return lax.axis_index("s") * NC + lax.axis_index("c")


def _iota16():
    return lax.iota(jnp.int32, L)


def _splat(x, dtype=jnp.int32):
    return jnp.full((L,), x, dtype=dtype)


def _first_lane(vec):
    return jnp.max(jnp.where(_iota16() == 0, vec, jnp.int32(-2147483648)))


def _k1_body(mem_hbm, lu_hbm, src_hbm, dst_hbm, t_hbm,
             mem_src_out, mem_dst_out, lu_src_out, lu_dst_out,
             nodes_out, posw_out, counts_out, lu_out,
             all_idx, rows_a, rows_b, lu_v, prio, nodes2d, posw2d, cnt128,
             t_v, lu_rng,
             sem_ga, sem_gb, sem_lu):
    wid = _wid()
    iota = _iota16()
    base_pos = wid * BPW
    HW = BPW // 2

    pltpu.sync_copy(src_hbm, all_idx.at[pl.ds(0, B)])
    pltpu.sync_copy(dst_hbm, all_idx.at[pl.ds(B, B)])
    pltpu.sync_copy(t_hbm, t_v)

    g0 = pltpu.async_copy(
        mem_hbm.at[all_idx.at[pl.ds(base_pos, HW)]], rows_a, sem_ga)
    g1 = pltpu.async_copy(
        mem_hbm.at[all_idx.at[pl.ds(base_pos + HW, HW)]], rows_b, sem_gb)
    src_idx = all_idx.at[pl.ds(base_pos, BPW)]
    dst_idx = all_idx.at[pl.ds(B + base_pos, BPW)]
    glu = pltpu.async_copy(lu_hbm.at[src_idx], lu_v, sem_lu)

    base_node = wid * RPW
    my_start = base_node
    my_len = jnp.where(wid == NW - 1, RPW_LAST, RPW)

    def init_prio(j, _):
        prio[pl.ds(j * L, L)] = _splat(-1)
        return 0
    lax.fori_loop(0, (RPW + 8 + L - 1) // L, init_prio, 0)

    def scan_step(v, _):
        idxv = all_idx[pl.ds(v * L, L)]
        loc = idxv - base_node
        inr = (loc >= 0) & (loc < my_len)
        _, last = plsc.scan_count(loc, mask=inr)
        win = last & inr
        pos = v * L + iota
        loc_safe = jnp.where(win, loc, 0)
        plsc.store_scatter(prio, [loc_safe], pos, mask=win)
        return 0

    lax.fori_loop(0, B2 // L, scan_step, 0)

    g0.wait()
    pltpu.sync_copy(rows_a, mem_src_out.at[pl.ds(base_pos, HW)])
    g2 = pltpu.async_copy(
        mem_hbm.at[all_idx.at[pl.ds(B + base_pos, HW)]], rows_a, sem_ga)
    g1.wait()
    pltpu.sync_copy(rows_b, mem_src_out.at[pl.ds(base_pos + HW, HW)])
    g3 = pltpu.async_copy(
        mem_hbm.at[all_idx.at[pl.ds(B + base_pos + HW, HW)]], rows_b, sem_gb)

    TAIL = RPW - RPW_LAST
    tail = my_len - TAIL
    pltpu.sync_copy(lu_hbm.at[pl.ds(my_start, RPW_LAST)],
                    lu_rng.at[pl.ds(0, RPW_LAST)])
    pltpu.sync_copy(lu_hbm.at[pl.ds(my_start + tail, TAIL)],
                    lu_rng.at[pl.ds(tail, TAIL)])

    def lu_step(j, _):
        pr = prio[pl.ds(j * L, L)]
        m = pr >= 0
        pm = pr & (B - 1)
        tv = plsc.load_gather(t_v, [pm])
        cur = lu_rng[pl.ds(j * L, L)]
        lu_rng[pl.ds(j * L, L)] = jnp.where(m, tv, cur)
        return 0
    lax.fori_loop(0, (RPW + 8) // L, lu_step, 0)

    pltpu.sync_copy(lu_rng.at[pl.ds(0, RPW_LAST)],
                    lu_out.at[pl.ds(my_start, RPW_LAST)])
    pltpu.sync_copy(lu_rng.at[pl.ds(tail, TAIL)],
                    lu_out.at[pl.ds(my_start + tail, TAIL)])

    glu.wait()
    pltpu.sync_copy(lu_v, lu_src_out.at[pl.ds(base_pos, BPW)])
    glu2 = pltpu.async_copy(lu_hbm.at[dst_idx], lu_v, sem_lu)

    def init_lists(j, _):
        r = j >> 3
        c = (j & 7) * L
        nodes2d[r, pl.ds(c, L)] = _splat(0)
        posw2d[r, pl.ds(c, L)] = _splat(0)
        return 0
    lax.fori_loop(0, LIST_CH * 8, init_lists, 0)

    def compact_step(j, cursor):
        pr = prio[pl.ds(j * L, L)]
        valid = pr >= 0
        node = base_node + j * L + iota
        incl = plsc.cumsum(jnp.where(valid, 1, 0))
        slot = cursor + incl - 1
        slot = jnp.maximum(slot, 0)
        srow = slot >> 7
        scol = slot & 127
        plsc.store_scatter(nodes2d, [srow, scol], node, mask=valid)
        plsc.store_scatter(posw2d, [srow, scol], pr, mask=valid)
        return cursor + jnp.max(incl)

    cnt = lax.fori_loop(0, (RPW + 8) // L, compact_step, jnp.int32(0))

    @pl.when(cnt > 0)
    def _fill():
        node0 = _first_lane(nodes2d[0, pl.ds(0, L)])
        pos0 = _first_lane(posw2d[0, pl.ds(0, L)])

        def fill_step(j, _):
            r = j >> 3
            c = (j & 7) * L
            slot = j * L + iota
            m = slot >= cnt
            cur_n = nodes2d[r, pl.ds(c, L)]
            cur_p = posw2d[r, pl.ds(c, L)]
            nodes2d[r, pl.ds(c, L)] = jnp.where(m, node0, cur_n)
            posw2d[r, pl.ds(c, L)] = jnp.where(m, pos0, cur_p)
            return 0
        lax.fori_loop(0, LIST_CH * 8, fill_step, 0)

    g2.wait()
    pltpu.sync_copy(rows_a, mem_dst_out.at[pl.ds(base_pos, HW)])
    g3.wait()
    pltpu.sync_copy(rows_b, mem_dst_out.at[pl.ds(base_pos + HW, HW)])
    glu2.wait()
    pltpu.sync_copy(lu_v, lu_dst_out.at[pl.ds(base_pos, BPW)])

    pltpu.sync_copy(nodes2d, nodes_out.at[wid])
    pltpu.sync_copy(posw2d, posw_out.at[wid])
    counts_v = jnp.full((L,), cnt, dtype=jnp.int32)
    for l in range(8):
        cnt128[pl.ds(l * L, L)] = counts_v
    pltpu.sync_copy(cnt128, counts_out.at[wid])


def _k1(memory, last_update, src, dst, t):
    out_type = (
        jax.ShapeDtypeStruct((B, D), jnp.float32),
        jax.ShapeDtypeStruct((B, D), jnp.float32),
        jax.ShapeDtypeStruct((B,), jnp.float32),
        jax.ShapeDtypeStruct((B,), jnp.float32),
        jax.ShapeDtypeStruct((NW, LIST_CH, 128), jnp.int32),
        jax.ShapeDtypeStruct((NW, LIST_CH, 128), jnp.int32),
        jax.ShapeDtypeStruct((NW, 128), jnp.int32),
        jax.ShapeDtypeStruct((N,), jnp.float32),
    )
    scratch = [
        pltpu.VMEM((B2,), jnp.int32),
        pltpu.VMEM((BPW // 2, D), jnp.float32),
        pltpu.VMEM((BPW // 2, D), jnp.float32),
        pltpu.VMEM((BPW,), jnp.float32),
        pltpu.VMEM((RPW + L, ), jnp.int32),
        pltpu.VMEM((LIST_CH, 128), jnp.int32),
        pltpu.VMEM((LIST_CH, 128), jnp.int32),
        pltpu.VMEM((128,), jnp.int32),
        pltpu.VMEM((B,), jnp.float32),
        pltpu.VMEM((RPW + 8,), jnp.float32),
        pltpu.SemaphoreType.DMA,
        pltpu.SemaphoreType.DMA,
        pltpu.SemaphoreType.DMA,
    ]
    return pl.kernel(
        _k1_body, out_type=out_type, mesh=_mesh(), scratch_types=scratch,
        compiler_params=pltpu.CompilerParams(needs_layout_passes=False),
    )(memory, last_update, src, dst, t)


_INV_2PI = 0.15915494309189535
_CW1 = 6.28125
_CW2 = 0.0019353071795864769
_COS_POLY = (1.0, -0.49999985098838806, 0.041666463017463684,
             -0.0013887732056900859, 2.4769053197815083e-05,
             -2.707544979330123e-07, 1.7243751981865785e-09)


def _fast_cos(x):
    n = jnp.round(x * _INV_2PI)
    r = x - n * _CW1 - n * _CW2
    u = r * r
    acc = jnp.float32(_COS_POLY[-1])
    for c in _COS_POLY[-2::-1]:
        acc = acc * u + jnp.float32(c)
    return acc


def _dense_body(t_ref, lus_ref, lud_ref, ms_ref, md_ref, ef_ref,
                tw_ref, tb_ref, w_own_ref, w_oth_ref, w_te_ref, w_ef_ref,
                w_hh_ref, bi_ref, bh_ref, out_ref):
    is_src = pl.program_id(1) == 0
    ms = ms_ref[...]
    md = md_ref[...]
    own = jnp.where(is_src, ms, md)
    oth = jnp.where(is_src, md, ms)
    lu = jnp.where(is_src, lus_ref[...], lud_ref[...])
    te = _fast_cos((t_ref[...] - lu) * tw_ref[...] + tb_ref[...])

    bf = jnp.bfloat16
    gi = (jnp.dot(own.astype(bf), w_own_ref[...], preferred_element_type=jnp.float32)
          + jnp.dot(oth.astype(bf), w_oth_ref[...], preferred_element_type=jnp.float32)
          + jnp.dot(te.astype(bf), w_te_ref[...], preferred_element_type=jnp.float32)
          + jnp.dot(ef_ref[...].astype(bf), w_ef_ref[...],
                    preferred_element_type=jnp.float32)
          + bi_ref[...])
    gh = jnp.dot(own.astype(bf), w_hh_ref[...], preferred_element_type=jnp.float32) \
        + bh_ref[...]
    r = jax.nn.sigmoid(gi[:, :D] + gh[:, :D])
    z = jax.nn.sigmoid(gi[:, D:2 * D] + gh[:, D:2 * D])
    n = jnp.tanh(gi[:, 2 * D:] + r * gh[:, 2 * D:])
    out_ref[...] = (1.0 - z) * n + z * own


def _dense_update(t2, lu_src, lu_dst, mem_src, mem_dst, edge_feat,
                  time_w, time_b, W_ih, W_hh, b_ih, b_hh):
    W_ihT = W_ih.T.astype(jnp.bfloat16)
    w_own = W_ihT[0:D]
    w_oth = W_ihT[D:2 * D]
    w_te = W_ihT[2 * D:3 * D]
    w_ef = W_ihT[3 * D:]
    w_hh = W_hh.T.astype(jnp.bfloat16)
    bi = b_ih.reshape(1, -1)
    bh = b_hh.reshape(1, -1)
    tw = time_w.reshape(1, -1)
    tb = time_b.reshape(1, -1)

    grid = (B // BM, 2)
    row_blk = pl.BlockSpec((BM, 1), lambda i, j: (i, 0))
    mat_blk = pl.BlockSpec((BM, D), lambda i, j: (i, 0))

    def full(a):
        return pl.BlockSpec(a.shape, lambda i, j: tuple(0 for _ in a.shape))

    out_spec = pl.BlockSpec((BM, D), lambda i, j: (j * (B // BM) + i, 0))
    return pl.pallas_call(
        _dense_body,
        grid=grid,
        in_specs=[row_blk, row_blk, row_blk, mat_blk, mat_blk, mat_blk,
                  full(tw), full(tb), full(w_own), full(w_oth), full(w_te),
                  full(w_ef), full(w_hh), full(bi), full(bh)],
        out_specs=out_spec,
        out_shape=jax.ShapeDtypeStruct((B2, D), jnp.float32),
    )(t2, lu_src, lu_dst, mem_src, mem_dst, edge_feat,
      tw, tb, w_own, w_oth, w_te, w_ef, w_hh, bi, bh)


def _k3_body(u_hbm, nodes_hbm, posw_hbm, counts_hbm,
             mem_ref,
             nodes2d, posw2d, rowbuf0, rowbuf1, tmp16,
             sem_ga, sem_gb, sem_sa, sem_sb):
    wid = _wid()

    pltpu.sync_copy(counts_hbm.at[wid], tmp16)
    cnt = jnp.max(tmp16[pl.ds(0, L)])
    pltpu.sync_copy(nodes_hbm.at[wid], nodes2d)
    pltpu.sync_copy(posw_hbm.at[wid], posw2d)

    @pl.when(cnt > 0)
    def _scatter():
        nb_ch = (cnt + 127) >> 7

        def sc_step(i, _):
            ca = 2 * i
            cb = jnp.minimum(2 * i + 1, nb_ch - 1)
            ga = pltpu.async_copy(u_hbm.at[posw2d.at[ca]], rowbuf0, sem_ga)
            gb = pltpu.async_copy(u_hbm.at[posw2d.at[cb]], rowbuf1, sem_gb)
            ga.wait()
            sa = pltpu.async_copy(rowbuf0, mem_ref.at[nodes2d.at[ca]], sem_sa)
            gb.wait()
            sb = pltpu.async_copy(rowbuf1, mem_ref.at[nodes2d.at[cb]], sem_sb)
            sa.wait()
            sb.wait()
            return 0

        lax.fori_loop(0, (nb_ch + 1) >> 1, sc_step, 0)


def _k3(u, nodes, posw, counts, mem_ref):
    scratch = [
        pltpu.VMEM((LIST_CH, 128), jnp.int32),
        pltpu.VMEM((LIST_CH, 128), jnp.int32),
        pltpu.VMEM((128, D), jnp.float32),
        pltpu.VMEM((128, D), jnp.float32),
        pltpu.VMEM((128,), jnp.int32),
    ] + [pltpu.SemaphoreType.DMA] * 4
    return pl.kernel(
        _k3_body, out_type=(), mesh=_mesh(), scratch_types=scratch,
        compiler_params=pltpu.CompilerParams(needs_layout_passes=False),
    )(u, nodes, posw, counts, mem_ref)


def kernel(memory, last_update, src, dst, t, edge_feat, time_w, time_b,
           W_ih, W_hh, b_ih, b_hh):
    mem_ref = jax.new_ref(memory)
    (mem_src, mem_dst, lu_src, lu_dst, nodes, posw, counts,
     new_last_update) = _k1(memory, last_update, src, dst, t)

    u = _dense_update(t.reshape(B, 1), lu_src.reshape(B, 1),
                      lu_dst.reshape(B, 1), mem_src, mem_dst,
                      edge_feat, time_w, time_b, W_ih, W_hh, b_ih, b_hh)

    _k3(u, nodes, posw, counts, mem_ref)
    return (mem_ref[...], new_last_update)
